# Initial kernel scaffold; baseline (speedup 1.0000x reference)
import jax
import jax.numpy as jnp
from jax.experimental import pallas as pl


def kernel(h_V, h_E, E_idx, mask_V, mask_attend, W1, b1, W2, b2, W3, b3, W11, b11, W12, b12, W13, b13, Wd_in, bd_in, Wd_out, bd_out, n1s, n1b, n2s, n2b, n3s, n3b):
    raise NotImplementedError("write your pallas kernel here")


# speedup vs baseline: 12.1347x; 12.1347x over previous
import functools

import jax
import jax.numpy as jnp
from jax import lax
from jax.experimental import pallas as pl
from jax.experimental.pallas import tpu as pltpu
from jax.experimental.pallas import tpu_sc as plsc

_B, _L, _K, _H = 2, 1024, 48, 128
_SCALE = 30.0
_TL = 128
_NB = _B * _L // _TL
_TLK = _TL * _K


def _gelu(x):
    return 0.5 * x * (1.0 + lax.erf(x * 0.7071067811865476))


def _lnorm(x, s, b):
    mu = jnp.mean(x, axis=-1, keepdims=True)
    xc = x - mu
    var = jnp.mean(xc * xc, axis=-1, keepdims=True)
    return xc * lax.rsqrt(var + 1e-5) * s + b


def _pre_body(hv_ref, w_ref, b1_ref, a1_ref, t1_ref):
    cat = jnp.dot(hv_ref[...], w_ref[...], preferred_element_type=jnp.float32)
    a1_ref[...] = cat[:, :_H] + b1_ref[...]
    t1_ref[...] = cat[:, _H:]


def _pre(hv, w1ab, b1r):
    return pl.pallas_call(
        _pre_body,
        out_shape=(
            jax.ShapeDtypeStruct((_B * _L, _H), jnp.float32),
            jax.ShapeDtypeStruct((_B * _L, _H), jnp.float32),
        ),
    )(hv, w1ab, b1r)


_GCH = 128


def _sc_gather(table, idx):
    info = plsc.get_sparse_core_info()
    nw = info.num_cores * info.num_subcores
    n = idx.shape[0]
    per_w = n // nw
    n_chunks = per_w // _GCH
    mesh = plsc.VectorSubcoreMesh(core_axis_name="c", subcore_axis_name="s")

    @functools.partial(
        pl.kernel,
        mesh=mesh,
        out_type=jax.ShapeDtypeStruct((n, _H), jnp.float32),
        scratch_types=[
            pltpu.VMEM((_GCH,), jnp.int32),
            pltpu.VMEM((_GCH, _H), jnp.float32),
            pltpu.SemaphoreType.DMA,
        ],
    )
    def gk(table_hbm, idx_hbm, out_hbm, idx_v, rows_v, sem):
        wid = lax.axis_index("s") * info.num_cores + lax.axis_index("c")
        base = wid * per_w

        def body(i, carry):
            off = base + i * _GCH
            pltpu.sync_copy(idx_hbm.at[pl.ds(off, _GCH)], idx_v)
            pltpu.async_copy(table_hbm.at[idx_v], rows_v, sem).wait()
            pltpu.sync_copy(rows_v, out_hbm.at[pl.ds(off, _GCH)])
            return carry

        lax.fori_loop(0, n_chunks, body, 0)

    return gk(table, idx)


def _m1_body(hv, a1, g1, he, ma, mv, w1c, w2, b2, w3, b3, wdi, bdi, wdo, bdo,
             n1s, n1b, n2s, n2b, w11ab, b11, hv2_o, a2_o, t2_o):
    e1 = jnp.dot(he[...], w1c[...], preferred_element_type=jnp.float32)
    a1b = jnp.broadcast_to(
        a1[...].reshape(_TL, 1, _H), (_TL, _K, _H)).reshape(_TLK, _H)
    x = _gelu(a1b + g1[...] + e1)
    y = _gelu(jnp.dot(x, w2[...], preferred_element_type=jnp.float32) + b2[...])
    z = jnp.dot(y, w3[...], preferred_element_type=jnp.float32) + b3[...]
    z = z * ma[...]
    dh = jnp.sum(z.reshape(_TL, _K, _H), axis=1) * (1.0 / _SCALE)
    v = _lnorm(hv[...] + dh, n1s[...], n1b[...])
    ff = jnp.dot(_gelu(jnp.dot(v, wdi[...], preferred_element_type=jnp.float32)
                       + bdi[...]), wdo[...],
                 preferred_element_type=jnp.float32) + bdo[...]
    v = _lnorm(v + ff, n2s[...], n2b[...]) * mv[...]
    hv2_o[...] = v
    cat = jnp.dot(v, w11ab[...], preferred_element_type=jnp.float32)
    a2_o[...] = cat[:, :_H] + b11[...]
    t2_o[...] = cat[:, _H:]


def _m1(hv, a1, g1, he, ma, mv, w1c, w2, b2, w3, b3, wdi, bdi, wdo, bdo,
        n1s, n1b, n2s, n2b, w11ab, b11):
    row = lambda i: (i, 0)
    whole = lambda i: (0, 0)
    return pl.pallas_call(
        _m1_body,
        grid=(_NB,),
        in_specs=[
            pl.BlockSpec((_TL, _H), row),
            pl.BlockSpec((_TL, _H), row),
            pl.BlockSpec((_TLK, _H), row),
            pl.BlockSpec((_TLK, _H), row),
            pl.BlockSpec((_TLK, 1), row),
            pl.BlockSpec((_TL, 1), row),
            pl.BlockSpec((_H, _H), whole),
            pl.BlockSpec((_H, _H), whole),
            pl.BlockSpec((1, _H), whole),
            pl.BlockSpec((_H, _H), whole),
            pl.BlockSpec((1, _H), whole),
            pl.BlockSpec((_H, 4 * _H), whole),
            pl.BlockSpec((1, 4 * _H), whole),
            pl.BlockSpec((4 * _H, _H), whole),
            pl.BlockSpec((1, _H), whole),
            pl.BlockSpec((1, _H), whole),
            pl.BlockSpec((1, _H), whole),
            pl.BlockSpec((1, _H), whole),
            pl.BlockSpec((1, _H), whole),
            pl.BlockSpec((_H, 2 * _H), whole),
            pl.BlockSpec((1, _H), whole),
        ],
        out_specs=[
            pl.BlockSpec((_TL, _H), row),
            pl.BlockSpec((_TL, _H), row),
            pl.BlockSpec((_TL, _H), row),
        ],
        out_shape=[
            jax.ShapeDtypeStruct((_B * _L, _H), jnp.float32),
            jax.ShapeDtypeStruct((_B * _L, _H), jnp.float32),
            jax.ShapeDtypeStruct((_B * _L, _H), jnp.float32),
        ],
    )(hv, a1, g1, he, ma, mv, w1c, w2, b2, w3, b3, wdi, bdi, wdo, bdo,
      n1s, n1b, n2s, n2b, w11ab, b11)


def _m2_body(a2, g2, he, w11c, w12, b12, w13, b13, n3s, n3b, out):
    e2 = jnp.dot(he[...], w11c[...], preferred_element_type=jnp.float32)
    a2b = jnp.broadcast_to(
        a2[...].reshape(_TL, 1, _H), (_TL, _K, _H)).reshape(_TLK, _H)
    x = _gelu(a2b + g2[...] + e2)
    y = _gelu(jnp.dot(x, w12[...], preferred_element_type=jnp.float32) + b12[...])
    z = jnp.dot(y, w13[...], preferred_element_type=jnp.float32) + b13[...]
    out[...] = _lnorm(he[...] + z, n3s[...], n3b[...])


def _m2(a2, g2, he, w11c, w12, b12, w13, b13, n3s, n3b):
    row = lambda i: (i, 0)
    whole = lambda i: (0, 0)
    return pl.pallas_call(
        _m2_body,
        grid=(_NB,),
        in_specs=[
            pl.BlockSpec((_TL, _H), row),
            pl.BlockSpec((_TLK, _H), row),
            pl.BlockSpec((_TLK, _H), row),
            pl.BlockSpec((_H, _H), whole),
            pl.BlockSpec((_H, _H), whole),
            pl.BlockSpec((1, _H), whole),
            pl.BlockSpec((_H, _H), whole),
            pl.BlockSpec((1, _H), whole),
            pl.BlockSpec((1, _H), whole),
            pl.BlockSpec((1, _H), whole),
        ],
        out_specs=pl.BlockSpec((_TLK, _H), row),
        out_shape=jax.ShapeDtypeStruct((_B * _L * _K, _H), jnp.float32),
    )(a2, g2, he, w11c, w12, b12, w13, b13, n3s, n3b)


def kernel(h_V, h_E, E_idx, mask_V, mask_attend, W1, b1, W2, b2, W3, b3,
           W11, b11, W12, b12, W13, b13, Wd_in, bd_in, Wd_out, bd_out,
           n1s, n1b, n2s, n2b, n3s, n3b):
    hv = h_V.reshape(_B * _L, _H)
    he = h_E.reshape(_B * _L * _K, _H)
    idx = (E_idx + (jnp.arange(_B, dtype=jnp.int32) * _L)[:, None, None])
    idx = idx.reshape(-1)
    ma = mask_attend.reshape(-1, 1)
    mv = mask_V.reshape(-1, 1)

    w1ab = jnp.concatenate([W1[:_H], W1[_H:2 * _H]], axis=1)
    w1c = W1[2 * _H:]
    w11ab = jnp.concatenate([W11[:_H], W11[_H:2 * _H]], axis=1)
    w11c = W11[2 * _H:]
    r = lambda v: v.reshape(1, -1)

    a1, t1 = _pre(hv, w1ab, r(b1))
    g1 = _sc_gather(t1, idx)
    hv2, a2, t2 = _m1(hv, a1, g1, he, ma, mv, w1c, W2, r(b2), W3, r(b3),
                      Wd_in, r(bd_in), Wd_out, r(bd_out),
                      r(n1s), r(n1b), r(n2s), r(n2b), w11ab, r(b11))
    g2 = _sc_gather(t2, idx)
    he2 = _m2(a2, g2, he, w11c, W12, r(b12), W13, r(b13), r(n3s), r(n3b))

    return hv2.reshape(_B, _L, _H), he2.reshape(_B, _L, _K, _H)
